# Initial kernel scaffold; baseline (speedup 1.0000x reference)
import jax
import jax.numpy as jnp
from jax.experimental import pallas as pl


def kernel(x, y):
    raise NotImplementedError("write your pallas kernel here")


# speedup vs baseline: 2.2218x; 2.2218x over previous
import jax
import jax.numpy as jnp
from jax import lax
from jax.experimental import pallas as pl
from jax.experimental.pallas import tpu as pltpu
from jax.experimental.pallas import tpu_sc as plsc

_NUM_CLASSES = 1000
_ALPHA = 0.3

_N = 16384
_D = 128
_NC = 2
_NS = 16
_NW = _NC * _NS
_RPT = _N // _NW
_CHUNK = 128
_NCHUNK = _RPT // _CHUNK
_CPAD = 1024
_ZROWS = _CPAD // _NS

_vmesh = plsc.VectorSubcoreMesh(core_axis_name="c", subcore_axis_name="s")


def _sc_stats_body(x_hbm, xsq_hbm, y_hbm, zeros_hbm, ones_hbm,
                   s_out, s2_out, cnt_out,
                   s_sh, s2_sh, cnt_sh, x_v, ones_v, idx_v):
  core = lax.axis_index("c")
  sid = lax.axis_index("s")
  wid = sid * _NC + core
  base = wid * _RPT

  zsl = pl.ds(sid * _ZROWS, _ZROWS)
  pltpu.sync_copy(zeros_hbm, s_sh.at[zsl])
  pltpu.sync_copy(zeros_hbm, s2_sh.at[zsl])
  pltpu.sync_copy(zeros_hbm, cnt_sh.at[zsl])

  pltpu.sync_copy(ones_hbm, ones_v)
  plsc.subcore_barrier()

  @pl.loop(0, _NCHUNK)
  def _chunk(j):
    off = base + j * _CHUNK
    pltpu.sync_copy(y_hbm.at[pl.ds(off, _CHUNK)], idx_v.at[0])
    pltpu.sync_copy(x_hbm.at[pl.ds(off, _CHUNK)], x_v)
    pltpu.sync_copy(x_v, s_sh.at[idx_v.at[0]], add=True)
    pltpu.sync_copy(xsq_hbm.at[pl.ds(off, _CHUNK)], x_v)
    pltpu.sync_copy(x_v, s2_sh.at[idx_v.at[0]], add=True)
    pltpu.sync_copy(ones_v, cnt_sh.at[idx_v.at[0]], add=True)

  plsc.subcore_barrier()

  pltpu.sync_copy(s_sh.at[zsl], s_out.at[core, zsl])
  pltpu.sync_copy(s2_sh.at[zsl], s2_out.at[core, zsl])
  pltpu.sync_copy(cnt_sh.at[zsl], cnt_out.at[core, zsl])


def _sc_gather_body(tab_hbm, ny_hbm, g_out, g_v, idx_v, sem):
  core = lax.axis_index("c")
  sid = lax.axis_index("s")
  wid = sid * _NC + core
  base = wid * _RPT

  @pl.loop(0, _NCHUNK)
  def _chunk(j):
    off = base + j * _CHUNK
    pltpu.sync_copy(ny_hbm.at[pl.ds(off, _CHUNK)], idx_v.at[0])
    pltpu.async_copy(tab_hbm.at[idx_v.at[0]], g_v, sem).wait()
    pltpu.sync_copy(g_v, g_out.at[pl.ds(off, _CHUNK)])


def _tc_square_body(x_ref, o_ref):
  x = x_ref[...]
  o_ref[...] = x * x


def _tc_finalize_body(s_ref, s2_ref, c_ref, o_ref):
  s = s_ref[0] + s_ref[1]
  s2 = s2_ref[0] + s2_ref[1]
  cnt = c_ref[0] + c_ref[1]
  mean = s / cnt
  var = (s2 - cnt * mean * mean) / (cnt - 1.0)
  std = jnp.sqrt(jnp.maximum(var, 0.0))
  o_ref[:, 0:_D] = mean
  o_ref[:, _D:2 * _D] = std


def _tc_combine_body(x_ref, e_ref, g_ref, o_ref):
  g = g_ref[...]
  noise = g[:, 0:_D] + g[:, _D:2 * _D] * e_ref[...]
  o_ref[...] = (1.0 - _ALPHA) * x_ref[...] + _ALPHA * noise


def _segment_stats(x, xsq, y, zeros, ones):
  sds = jax.ShapeDtypeStruct((_NC, _CPAD, _D), jnp.float32)
  k = pl.kernel(
      _sc_stats_body,
      out_type=(sds, sds, sds),
      mesh=_vmesh,
      scratch_types=[
          pltpu.VMEM_SHARED((_CPAD, _D), jnp.float32),
          pltpu.VMEM_SHARED((_CPAD, _D), jnp.float32),
          pltpu.VMEM_SHARED((_CPAD, _D), jnp.float32),
          pltpu.VMEM((_CHUNK, _D), jnp.float32),
          pltpu.VMEM((_CHUNK, _D), jnp.float32),
          pltpu.VMEM((1, _CHUNK), jnp.int32),
      ],
  )
  return k(x, xsq, y, zeros, ones)


def _gather_rows(tab, ny):
  k = pl.kernel(
      _sc_gather_body,
      out_type=jax.ShapeDtypeStruct((_N, 2 * _D), jnp.float32),
      mesh=_vmesh,
      scratch_types=[
          pltpu.VMEM((_CHUNK, 2 * _D), jnp.float32),
          pltpu.VMEM((1, _CHUNK), jnp.int32),
          pltpu.SemaphoreType.DMA,
      ],
  )
  return k(tab, ny)


def _resample(y, key):
  k1, k2 = jax.random.split(key)
  perm = jax.random.permutation(k1, y.shape[0])
  new_y = y[perm]

  def cond(state):
    ny, _ = state
    return jnp.any(ny == y)

  def body(state):
    ny, k = state
    k, sub = jax.random.split(k)
    rand = jax.random.randint(sub, y.shape, 0, _NUM_CLASSES).astype(y.dtype)
    ny = jnp.where(ny == y, rand, ny)
    return (ny, k)

  new_y, _ = jax.lax.while_loop(cond, body, (new_y, k2))
  return new_y


def kernel(x, y):
  key = jax.random.key(42)
  k_perm, k_noise = jax.random.split(key)
  new_y = _resample(y, k_perm)
  eps = jax.random.normal(k_noise, x.shape, dtype=x.dtype)

  blk = 1024
  xsq = pl.pallas_call(
      _tc_square_body,
      grid=(_N // blk,),
      in_specs=[pl.BlockSpec((blk, _D), lambda i: (i, 0))],
      out_specs=pl.BlockSpec((blk, _D), lambda i: (i, 0)),
      out_shape=jax.ShapeDtypeStruct((_N, _D), jnp.float32),
  )(x)

  zeros = jnp.zeros((_ZROWS, _D), jnp.float32)
  ones = jnp.ones((_CHUNK, _D), jnp.float32)
  s_p, s2_p, cnt_p = _segment_stats(x, xsq, y, zeros, ones)

  tab = pl.pallas_call(
      _tc_finalize_body,
      out_shape=jax.ShapeDtypeStruct((_CPAD, 2 * _D), jnp.float32),
  )(s_p, s2_p, cnt_p)

  gmgs = _gather_rows(tab, new_y)

  out = pl.pallas_call(
      _tc_combine_body,
      grid=(_N // blk,),
      in_specs=[
          pl.BlockSpec((blk, _D), lambda i: (i, 0)),
          pl.BlockSpec((blk, _D), lambda i: (i, 0)),
          pl.BlockSpec((blk, 2 * _D), lambda i: (i, 0)),
      ],
      out_specs=pl.BlockSpec((blk, _D), lambda i: (i, 0)),
      out_shape=jax.ShapeDtypeStruct((_N, _D), jnp.float32),
  )(x, eps, gmgs)

  return (out, new_y)
